# baseline (speedup 1.0000x reference)
import jax
import jax.numpy as jnp
from jax.experimental import pallas as pl
from jax.experimental.pallas import tpu as pltpu

_B = 2
_T = 196
_DIM = 768
_HEADS = 12
_DH = _DIM // _HEADS
_TOPK = 49


def _gelu_exact(x):
    return 0.5 * x * (1.0 + jax.lax.erf(x * (2.0 ** -0.5)))


def _split_bf16(a):
    ah = a.astype(jnp.bfloat16)
    al = (a - ah.astype(jnp.float32)).astype(jnp.bfloat16)
    return ah, al


def _dot3(a, b, dims):
    ah, al = _split_bf16(a)
    bh, bl = _split_bf16(b)

    def d(u, w):
        return jax.lax.dot_general(u, w, dims, preferred_element_type=jnp.float32)

    return d(ah, bh) + d(ah, bl) + d(al, bh)

_TN = (((0,), (0,)), ((), ()))
_NN = (((1,), (0,)), ((), ()))
_NT = (((1,), (1,)), ((), ()))


def _head_contrib(adj, xh, wg, wo, upper_bf):
    msg = _dot3(adj, xh, _NN) + xh
    qkv = _dot3(msg, wg, _NN)
    q = qkv[:, 0 * _DH:1 * _DH]
    k = qkv[:, 1 * _DH:2 * _DH]
    v = qkv[:, 2 * _DH:3 * _DH]

    min32 = jnp.int32(-(2 ** 31))
    bits = jax.lax.bitcast_convert_type(adj.T, jnp.int32)
    sT = bits ^ (jax.lax.shift_right_arithmetic(bits, 31) & jnp.int32(0x7FFFFFFF))

    t_u = jnp.zeros((1, _T), jnp.int32)
    for i in range(32):
        t_try = t_u | jnp.int32((1 << (31 - i)) - (2 ** 32 if i == 0 else 0))
        thr = t_try ^ min32
        ge = jnp.where(sT >= thr, 1.0, 0.0)
        cnt = jnp.sum(ge, axis=0, keepdims=True)
        t_u = jnp.where(cnt >= _TOPK, t_try, t_u)
    thr_s = t_u ^ min32

    gtT = sT > thr_s
    eqT = sT == thr_s
    cnt_gt = jnp.sum(jnp.where(gtT, 1.0, 0.0), axis=0, keepdims=True)
    exclT = jax.lax.dot_general(upper_bf, eqT.astype(jnp.bfloat16), _TN,
                                preferred_element_type=jnp.float32)
    keepT = eqT & (exclT < (_TOPK - cnt_gt))
    maskT = gtT | keepT

    scale = _DIM ** -0.5
    scoresT = _dot3(k, q * scale, _NT)
    scoresT = jnp.where(maskT, scoresT, -1e30)
    m = jnp.max(scoresT, axis=0, keepdims=True)
    e = jnp.exp(scoresT - m)
    p = e * (1.0 / jnp.sum(e, axis=0, keepdims=True))
    o = _dot3(p, v, _TN)

    return _dot3(_gelu_exact(o), wo, _NN)


_G = 2
_HG = _HEADS // _G


def _batch_kernel(adj_ref, x_ref, wg_ref, wv_ref, out_ref):
    rows = jax.lax.broadcasted_iota(jnp.int32, (_T, _T), 0)
    cols = jax.lax.broadcasted_iota(jnp.int32, (_T, _T), 1)
    upper_bf = (rows < cols).astype(jnp.bfloat16)

    g = pl.program_id(1)
    wg = wg_ref[...]
    x = x_ref[0]

    acc = None
    for h in range(_HG):
        contrib = _head_contrib(adj_ref[h], x[:, h * _DH:(h + 1) * _DH], wg,
                                wv_ref[h * _DH:(h + 1) * _DH, :], upper_bf)
        acc = contrib if acc is None else acc + contrib

    @pl.when(g == 0)
    def _():
        out_ref[0] = acc

    @pl.when(g != 0)
    def _():
        out_ref[0] = out_ref[0] + acc


def kernel(x, adj, rep_adj_dis, W_gin, W_v):
    del rep_adj_dis

    out = pl.pallas_call(
        _batch_kernel,
        grid=(_B, _G),
        in_specs=[
            pl.BlockSpec((_HG, _T, _T), lambda b, g: (b * _G + g, 0, 0)),
            pl.BlockSpec((1, _T, _HG * _DH), lambda b, g: (b, 0, g)),
            pl.BlockSpec((_DH, 3 * _DH), lambda b, g: (0, 0)),
            pl.BlockSpec((_HG * _DH, _DIM), lambda b, g: (g, 0)),
        ],
        out_specs=pl.BlockSpec((1, _T, _DIM), lambda b, g: (b, 0, 0)),
        out_shape=jax.ShapeDtypeStruct((_B, _T, _DIM), jnp.float32),
    )(adj, x, W_gin, W_v)
    return out


# speedup vs baseline: 1.1315x; 1.1315x over previous
import jax
import jax.numpy as jnp
from jax.experimental import pallas as pl
from jax.experimental.pallas import tpu as pltpu

_B = 2
_T = 196
_DIM = 768
_HEADS = 12
_DH = _DIM // _HEADS
_TOPK = 49


def _gelu_exact(x):
    return 0.5 * x * (1.0 + jax.lax.erf(x * (2.0 ** -0.5)))


def _split_bf16(a):
    ah = a.astype(jnp.bfloat16)
    al = (a - ah.astype(jnp.float32)).astype(jnp.bfloat16)
    return ah, al


def _dot3(a, b, dims):
    ah, al = _split_bf16(a)
    bh, bl = _split_bf16(b)

    def d(u, w):
        return jax.lax.dot_general(u, w, dims, preferred_element_type=jnp.float32)

    return d(ah, bh) + d(ah, bl) + d(al, bh)

_TN = (((0,), (0,)), ((), ()))
_NN = (((1,), (0,)), ((), ()))
_NT = (((1,), (1,)), ((), ()))


def _head_contrib(adj, xh, wg, upper_bf):
    msg = _dot3(adj, xh, _NN) + xh
    qkv = _dot3(msg, wg, _NN)
    q = qkv[:, 0 * _DH:1 * _DH]
    k = qkv[:, 1 * _DH:2 * _DH]
    v = qkv[:, 2 * _DH:3 * _DH]

    min32 = jnp.int32(-(2 ** 31))
    bits = jax.lax.bitcast_convert_type(adj.T, jnp.int32)
    sT = bits ^ (jax.lax.shift_right_arithmetic(bits, 31) & jnp.int32(0x7FFFFFFF))

    t_u = jnp.zeros((1, _T), jnp.int32)
    for i in range(32):
        t_try = t_u | jnp.int32((1 << (31 - i)) - (2 ** 32 if i == 0 else 0))
        thr = t_try ^ min32
        ge = jnp.where(sT >= thr, 1.0, 0.0)
        cnt = jnp.sum(ge, axis=0, keepdims=True)
        t_u = jnp.where(cnt >= _TOPK, t_try, t_u)
    thr_s = t_u ^ min32

    gtT = sT > thr_s
    eqT = sT == thr_s
    cnt_gt = jnp.sum(jnp.where(gtT, 1.0, 0.0), axis=0, keepdims=True)
    exclT = jax.lax.dot_general(upper_bf, eqT.astype(jnp.bfloat16), _TN,
                                preferred_element_type=jnp.float32)
    keepT = eqT & (exclT < (_TOPK - cnt_gt))
    maskT = gtT | keepT

    scale = _DIM ** -0.5
    scoresT = _dot3(k, q * scale, _NT)
    scoresT = jnp.where(maskT, scoresT, -1e30)
    m = jnp.max(scoresT, axis=0, keepdims=True)
    e = jnp.exp(scoresT - m)
    p = e * (1.0 / jnp.sum(e, axis=0, keepdims=True))
    o = _dot3(p, v, _TN)

    return _gelu_exact(o)


def _batch_kernel(adj_ref, x_ref, wg_ref, wv_ref, out_ref):
    rows = jax.lax.broadcasted_iota(jnp.int32, (_T, _T), 0)
    cols = jax.lax.broadcasted_iota(jnp.int32, (_T, _T), 1)
    upper_bf = (rows < cols).astype(jnp.bfloat16)

    wg = wg_ref[...]
    x = x_ref[0]

    gs = [_head_contrib(adj_ref[h], x[:, h * _DH:(h + 1) * _DH], wg, upper_bf)
          for h in range(_HEADS)]
    gout = jnp.concatenate(gs, axis=1)
    out_ref[0] = _dot3(gout, wv_ref[...], _NN)


def kernel(x, adj, rep_adj_dis, W_gin, W_v):
    del rep_adj_dis

    out = pl.pallas_call(
        _batch_kernel,
        grid=(_B,),
        in_specs=[
            pl.BlockSpec((_HEADS, _T, _T), lambda b: (b, 0, 0)),
            pl.BlockSpec((1, _T, _DIM), lambda b: (b, 0, 0)),
            pl.BlockSpec((_DH, 3 * _DH), lambda b: (0, 0)),
            pl.BlockSpec((_DIM, _DIM), lambda b: (0, 0)),
        ],
        out_specs=pl.BlockSpec((1, _T, _DIM), lambda b: (b, 0, 0)),
        out_shape=jax.ShapeDtypeStruct((_B, _T, _DIM), jnp.float32),
    )(adj, x, W_gin, W_v)
    return out


# speedup vs baseline: 1.1320x; 1.0005x over previous
import jax
import jax.numpy as jnp
from jax.experimental import pallas as pl

_B = 2
_T = 196
_DIM = 768
_HEADS = 12
_DH = _DIM // _HEADS
_TOPK = 49


def _gelu_exact(x):
    return 0.5 * x * (1.0 + jax.lax.erf(x * (2.0 ** -0.5)))


def _split_bf16(a):
    ah = a.astype(jnp.bfloat16)
    al = (a - ah.astype(jnp.float32)).astype(jnp.bfloat16)
    return ah, al


def _dot3(a, b, dims):
    ah, al = _split_bf16(a)
    bh, bl = _split_bf16(b)

    def d(u, w):
        return jax.lax.dot_general(u, w, dims, preferred_element_type=jnp.float32)

    return d(ah, bh) + d(ah, bl) + d(al, bh)

_TN = (((0,), (0,)), ((), ()))
_NN = (((1,), (0,)), ((), ()))
_NT = (((1,), (1,)), ((), ()))


def _head_contrib(adj, xh, wg, upper_bf):
    msg = _dot3(adj, xh, _NN) + xh
    qkv = _dot3(msg, wg, _NN)
    q = qkv[:, 0 * _DH:1 * _DH]
    k = qkv[:, 1 * _DH:2 * _DH]
    v = qkv[:, 2 * _DH:3 * _DH]

    min32 = jnp.int32(-(2 ** 31))
    bits = jax.lax.bitcast_convert_type(adj.T, jnp.int32)
    sT = bits ^ (jax.lax.shift_right_arithmetic(bits, 31) & jnp.int32(0x7FFFFFFF))

    t_u = jnp.zeros((1, _T), jnp.int32)
    for i in range(32):
        t_try = t_u | jnp.int32((1 << (31 - i)) - (2 ** 32 if i == 0 else 0))
        thr = t_try ^ min32
        ge = jnp.where(sT >= thr, 1.0, 0.0)
        cnt = jnp.sum(ge, axis=0, keepdims=True)
        t_u = jnp.where(cnt >= _TOPK, t_try, t_u)
    thr_s = t_u ^ min32

    gtT = sT > thr_s
    eqT = sT == thr_s
    cnt_gt = jnp.sum(jnp.where(gtT, 1.0, 0.0), axis=0, keepdims=True)
    exclT = jax.lax.dot_general(upper_bf, eqT.astype(jnp.bfloat16), _TN,
                                preferred_element_type=jnp.float32)
    keepT = eqT & (exclT < (_TOPK - cnt_gt))
    maskT = gtT | keepT

    scale = _DIM ** -0.5
    scoresT = _dot3(k, q * scale, _NT)
    scoresT = jnp.where(maskT, scoresT, -1e30)
    m = jnp.max(scoresT, axis=0, keepdims=True)
    e = jnp.exp(scoresT - m)
    p = e * (1.0 / jnp.sum(e, axis=0, keepdims=True))
    o = _dot3(p, v, _TN)

    return _gelu_exact(o)


def _batch_kernel(adj_ref, x_ref, wg_ref, wv_ref, out_ref):
    rows = jax.lax.broadcasted_iota(jnp.int32, (_T, _T), 0)
    cols = jax.lax.broadcasted_iota(jnp.int32, (_T, _T), 1)
    upper_bf = (rows < cols).astype(jnp.bfloat16)

    wg = wg_ref[...]
    x = x_ref[0]

    gs = [_head_contrib(adj_ref[h], x[:, h * _DH:(h + 1) * _DH], wg, upper_bf)
          for h in range(_HEADS)]
    gout = jnp.concatenate(gs, axis=1)
    out_ref[0] = _dot3(gout, wv_ref[...], _NN)


def kernel(x, adj, rep_adj_dis, W_gin, W_v):
    del rep_adj_dis

    out = pl.pallas_call(
        _batch_kernel,
        grid=(_B,),
        in_specs=[
            pl.BlockSpec((_HEADS, _T, _T), lambda b: (b, 0, 0)),
            pl.BlockSpec((1, _T, _DIM), lambda b: (b, 0, 0)),
            pl.BlockSpec((_DH, 3 * _DH), lambda b: (0, 0)),
            pl.BlockSpec((_DIM, _DIM), lambda b: (0, 0)),
        ],
        out_specs=pl.BlockSpec((1, _T, _DIM), lambda b: (b, 0, 0)),
        out_shape=jax.ShapeDtypeStruct((_B, _T, _DIM), jnp.float32),
    )(adj, x, W_gin, W_v)
    return out


# speedup vs baseline: 1.2199x; 1.0777x over previous
import jax
import jax.numpy as jnp
from jax.experimental import pallas as pl

_B = 2
_T = 196
_DIM = 768
_HEADS = 12
_DH = _DIM // _HEADS
_TOPK = 49
_PK = 3


def _gelu_exact(x):
    return 0.5 * x * (1.0 + jax.lax.erf(x * (2.0 ** -0.5)))


def _split_bf16(a):
    ah = a.astype(jnp.bfloat16)
    al = (a - ah.astype(jnp.float32)).astype(jnp.bfloat16)
    return ah, al


def _dot3(a, b, dims):
    ah, al = _split_bf16(a)
    bh, bl = _split_bf16(b)

    def d(u, w):
        return jax.lax.dot_general(u, w, dims, preferred_element_type=jnp.float32)

    return d(ah, bh) + d(ah, bl) + d(al, bh)

_TN = (((0,), (0,)), ((), ()))
_NN = (((1,), (0,)), ((), ()))
_NT = (((1,), (1,)), ((), ()))


def _keys(a):
    bits = jax.lax.bitcast_convert_type(a, jnp.int32)
    return bits ^ (jax.lax.shift_right_arithmetic(bits, 31)
                   & jnp.int32(0x7FFFFFFF))


def _radix_topk_threshold(sTg):
    min32 = jnp.int32(-(2 ** 31))
    lanes = sTg.shape[1]
    t_u = jnp.zeros((1, lanes), jnp.int32)
    for i in range(32):
        t_try = t_u | jnp.int32((1 << (31 - i)) - (2 ** 32 if i == 0 else 0))
        thr = t_try ^ min32
        ge = jnp.where(sTg >= thr, 1.0, 0.0)
        cnt = jnp.sum(ge, axis=0, keepdims=True)
        t_u = jnp.where(cnt >= _TOPK, t_try, t_u)
    return t_u ^ min32


def _head_contrib(adj, xh, wg, upper_bf, thr_s):
    msg = _dot3(adj, xh, _NN) + xh
    qkv = _dot3(msg, wg, _NN)
    q = qkv[:, 0 * _DH:1 * _DH]
    k = qkv[:, 1 * _DH:2 * _DH]
    v = qkv[:, 2 * _DH:3 * _DH]

    sT = _keys(adj.T)

    gtT = sT > thr_s
    eqT = sT == thr_s
    cnt_gt = jnp.sum(jnp.where(gtT, 1.0, 0.0), axis=0, keepdims=True)
    exclT = jax.lax.dot_general(upper_bf, eqT.astype(jnp.bfloat16), _TN,
                                preferred_element_type=jnp.float32)
    keepT = eqT & (exclT < (_TOPK - cnt_gt))
    maskT = gtT | keepT

    scale = _DIM ** -0.5
    scoresT = _dot3(k, q * scale, _NT)
    scoresT = jnp.where(maskT, scoresT, -1e30)
    m = jnp.max(scoresT, axis=0, keepdims=True)
    e = jnp.exp(scoresT - m)
    p = e * (1.0 / jnp.sum(e, axis=0, keepdims=True))
    o = _dot3(p, v, _TN)

    return _gelu_exact(o)


def _batch_kernel(adj_ref, x_ref, wg_ref, wv_ref, out_ref):
    rows = jax.lax.broadcasted_iota(jnp.int32, (_T, _T), 0)
    cols = jax.lax.broadcasted_iota(jnp.int32, (_T, _T), 1)
    upper_bf = (rows < cols).astype(jnp.bfloat16)

    wg = wg_ref[...]
    x = x_ref[0]

    thrs = []
    for g in range(_HEADS // _PK):
        adjg = adj_ref[_PK * g:_PK * (g + 1)]
        sTg = _keys(adjg.reshape(_PK * _T, _T).T)
        thrg = _radix_topk_threshold(sTg)
        for j in range(_PK):
            thrs.append(jax.lax.slice(thrg, (0, j * _T), (1, (j + 1) * _T)))

    gs = [_head_contrib(adj_ref[h], x[:, h * _DH:(h + 1) * _DH], wg, upper_bf,
                        thrs[h])
          for h in range(_HEADS)]
    gout = jnp.concatenate(gs, axis=1)
    out_ref[0] = _dot3(gout, wv_ref[...], _NN)


def kernel(x, adj, rep_adj_dis, W_gin, W_v):
    del rep_adj_dis

    out = pl.pallas_call(
        _batch_kernel,
        grid=(_B,),
        in_specs=[
            pl.BlockSpec((_HEADS, _T, _T), lambda b: (b, 0, 0)),
            pl.BlockSpec((1, _T, _DIM), lambda b: (b, 0, 0)),
            pl.BlockSpec((_DH, 3 * _DH), lambda b: (0, 0)),
            pl.BlockSpec((_DIM, _DIM), lambda b: (0, 0)),
        ],
        out_specs=pl.BlockSpec((1, _T, _DIM), lambda b: (b, 0, 0)),
        out_shape=jax.ShapeDtypeStruct((_B, _T, _DIM), jnp.float32),
    )(adj, x, W_gin, W_v)
    return out


# speedup vs baseline: 1.2397x; 1.0162x over previous
import jax
import jax.numpy as jnp
from jax.experimental import pallas as pl

_B = 2
_T = 196
_DIM = 768
_HEADS = 12
_DH = _DIM // _HEADS
_TOPK = 49
_PK = 12


def _gelu_exact(x):
    return 0.5 * x * (1.0 + jax.lax.erf(x * (2.0 ** -0.5)))


def _split_bf16(a):
    ah = a.astype(jnp.bfloat16)
    al = (a - ah.astype(jnp.float32)).astype(jnp.bfloat16)
    return ah, al


def _dot3(a, b, dims):
    ah, al = _split_bf16(a)
    bh, bl = _split_bf16(b)

    def d(u, w):
        return jax.lax.dot_general(u, w, dims, preferred_element_type=jnp.float32)

    return d(ah, bh) + d(ah, bl) + d(al, bh)

_TN = (((0,), (0,)), ((), ()))
_NN = (((1,), (0,)), ((), ()))
_NT = (((1,), (1,)), ((), ()))


def _keys(a):
    bits = jax.lax.bitcast_convert_type(a, jnp.int32)
    return bits ^ (jax.lax.shift_right_arithmetic(bits, 31)
                   & jnp.int32(0x7FFFFFFF))


def _radix_topk_threshold(sTg):
    min32 = jnp.int32(-(2 ** 31))
    lanes = sTg.shape[1]
    t_u = jnp.zeros((1, lanes), jnp.int32)
    for i in range(32):
        t_try = t_u | jnp.int32((1 << (31 - i)) - (2 ** 32 if i == 0 else 0))
        thr = t_try ^ min32
        ge = jnp.where(sTg >= thr, 1.0, 0.0)
        cnt = jnp.sum(ge, axis=0, keepdims=True)
        t_u = jnp.where(cnt >= _TOPK, t_try, t_u)
    return t_u ^ min32


def _head_contrib(adj, xh, wg, upper_bf, thr_s):
    msg = _dot3(adj, xh, _NN) + xh
    qkv = _dot3(msg, wg, _NN)
    q = qkv[:, 0 * _DH:1 * _DH]
    k = qkv[:, 1 * _DH:2 * _DH]
    v = qkv[:, 2 * _DH:3 * _DH]

    sT = _keys(adj.T)

    gtT = sT > thr_s
    eqT = sT == thr_s
    cnt_gt = jnp.sum(jnp.where(gtT, 1.0, 0.0), axis=0, keepdims=True)
    exclT = jax.lax.dot_general(upper_bf, eqT.astype(jnp.bfloat16), _TN,
                                preferred_element_type=jnp.float32)
    keepT = eqT & (exclT < (_TOPK - cnt_gt))
    maskT = gtT | keepT

    scale = _DIM ** -0.5
    scoresT = _dot3(k, q * scale, _NT)
    scoresT = jnp.where(maskT, scoresT, -1e30)
    m = jnp.max(scoresT, axis=0, keepdims=True)
    e = jnp.exp(scoresT - m)
    p = e * (1.0 / jnp.sum(e, axis=0, keepdims=True))
    o = _dot3(p, v, _TN)

    return _gelu_exact(o)


def _batch_kernel(adj_ref, x_ref, wg_ref, wv_ref, out_ref):
    rows = jax.lax.broadcasted_iota(jnp.int32, (_T, _T), 0)
    cols = jax.lax.broadcasted_iota(jnp.int32, (_T, _T), 1)
    upper_bf = (rows < cols).astype(jnp.bfloat16)

    wg = wg_ref[...]
    x = x_ref[0]

    thrs = []
    for g in range(_HEADS // _PK):
        adjg = adj_ref[_PK * g:_PK * (g + 1)]
        sTg = _keys(adjg.reshape(_PK * _T, _T).T)
        thrg = _radix_topk_threshold(sTg)
        for j in range(_PK):
            thrs.append(jax.lax.slice(thrg, (0, j * _T), (1, (j + 1) * _T)))

    gs = [_head_contrib(adj_ref[h], x[:, h * _DH:(h + 1) * _DH], wg, upper_bf,
                        thrs[h])
          for h in range(_HEADS)]
    gout = jnp.concatenate(gs, axis=1)
    out_ref[0] = _dot3(gout, wv_ref[...], _NN)


def kernel(x, adj, rep_adj_dis, W_gin, W_v):
    del rep_adj_dis

    out = pl.pallas_call(
        _batch_kernel,
        grid=(_B,),
        in_specs=[
            pl.BlockSpec((_HEADS, _T, _T), lambda b: (b, 0, 0)),
            pl.BlockSpec((1, _T, _DIM), lambda b: (b, 0, 0)),
            pl.BlockSpec((_DH, 3 * _DH), lambda b: (0, 0)),
            pl.BlockSpec((_DIM, _DIM), lambda b: (0, 0)),
        ],
        out_specs=pl.BlockSpec((1, _T, _DIM), lambda b: (b, 0, 0)),
        out_shape=jax.ShapeDtypeStruct((_B, _T, _DIM), jnp.float32),
    )(adj, x, W_gin, W_v)
    return out
